# baseline (speedup 1.0000x reference)
import functools

import jax
import jax.numpy as jnp
from jax import lax
from jax.experimental import pallas as pl
from jax.experimental.pallas import tpu as pltpu
from jax.experimental.pallas import tpu_sc as plsc

N_NODES = 10000
DIM = 128
NEG_SLOPE = 0.2

NC = 2
HALF = 5120
TRASH = 5120
AGG_ROWS = 5128
A_SL = HALF // 16
NS = 16
NW = NS
E_TOT = 320000 + N_NODES
CHUNK = 32
SBC = 6
SB_CNT = 108
SB_E = SBC * CHUNK
T_PER = SB_E * SB_CNT
E_PAD = T_PER * NS
N_PAD = E_PAD - E_TOT
D_PAD = 10240
A_PAD = 10112
R_SL = A_PAD // NS
NBUF = 4

ROW_BLK = 1000


def _tc_prep_body(x_ref, w_ref, asv_ref, adv_ref, xw_ref, asr_ref, adr_ref):
    xw = jnp.dot(x_ref[...], w_ref[...], preferred_element_type=jnp.float32)
    xw_ref[...] = xw
    asr_ref[...] = jnp.sum(xw * asv_ref[...], axis=1, keepdims=True)
    adr_ref[...] = jnp.sum(xw * adv_ref[...], axis=1, keepdims=True)


_tc_prep = pl.pallas_call(
    _tc_prep_body,
    grid=(N_NODES // ROW_BLK,),
    in_specs=[
        pl.BlockSpec((ROW_BLK, DIM), lambda i: (i, 0)),
        pl.BlockSpec((DIM, DIM), lambda i: (0, 0)),
        pl.BlockSpec((1, DIM), lambda i: (0, 0)),
        pl.BlockSpec((1, DIM), lambda i: (0, 0)),
    ],
    out_specs=[
        pl.BlockSpec((ROW_BLK, DIM), lambda i: (i, 0)),
        pl.BlockSpec((ROW_BLK, 1), lambda i: (i, 0)),
        pl.BlockSpec((ROW_BLK, 1), lambda i: (i, 0)),
    ],
    out_shape=[
        jax.ShapeDtypeStruct((N_NODES, DIM), jnp.float32),
        jax.ShapeDtypeStruct((N_NODES, 1), jnp.float32),
        jax.ShapeDtypeStruct((N_NODES, 1), jnp.float32),
    ],
)


def _tc_post_body(a0_ref, dp_ref, b_ref, out_ref):
    i = pl.program_id(0)
    dp = dp_ref[:, pl.ds(i * 128, 128)]
    d = jnp.sum(dp, axis=0)[:, None]
    h = a0_ref[...] / (d + 1e-16) + b_ref[...]
    out_ref[...] = h * jax.nn.sigmoid(h)


_tc_post = pl.pallas_call(
    _tc_post_body,
    grid=(A_PAD // 128,),
    in_specs=[
        pl.BlockSpec((128, DIM), lambda i: (i, 0)),
        pl.BlockSpec((NC * NS, A_PAD), lambda i: (0, 0)),
        pl.BlockSpec((1, DIM), lambda i: (0, 0)),
    ],
    out_specs=pl.BlockSpec((128, DIM), lambda i: (i, 0)),
    out_shape=jax.ShapeDtypeStruct((A_PAD, DIM), jnp.float32),
)


@functools.cache
def _get_sc_edge():
  return pl.kernel(
    _sc_edge_body,
    out_type=[
        jax.ShapeDtypeStruct((NC, HALF, DIM), jnp.float32),
        jax.ShapeDtypeStruct((NC * NS, A_PAD), jnp.float32),
    ],
    mesh=plsc.VectorSubcoreMesh(core_axis_name="c", subcore_axis_name="s",
                                num_cores=NC, num_subcores=NS),
    compiler_params=pltpu.CompilerParams(needs_layout_passes=False),
    scratch_types=[
        pltpu.VMEM((N_NODES,), jnp.float32),
        pltpu.VMEM((N_NODES,), jnp.float32),
        pltpu.VMEM((A_PAD,), jnp.float32),
        pltpu.VMEM((2 * SB_E,), jnp.int32),
        pltpu.VMEM((SBC, CHUNK), jnp.int32),
        pltpu.VMEM((SBC, CHUNK), jnp.int32),
        pltpu.VMEM((SB_E,), jnp.float32),
        pltpu.VMEM((NBUF * CHUNK, DIM), jnp.float32),
        pltpu.VMEM_SHARED((AGG_ROWS, DIM), jnp.float32),
        pltpu.SemaphoreType.DMA,
        pltpu.SemaphoreType.DMA,
        pltpu.SemaphoreType.DMA,
        pltpu.SemaphoreType.DMA,
        pltpu.SemaphoreType.DMA,
        pltpu.SemaphoreType.DMA,
        pltpu.SemaphoreType.DMA,
        pltpu.SemaphoreType.DMA,
        pltpu.SemaphoreType.DMA,
        pltpu.SemaphoreType.DMA,
        pltpu.SemaphoreType.DMA,
    ],
  )


def _sc_edge_body(xw_hbm, asrc_hbm, adst_hbm, src_hbm, dst6_hbm,
                  shift_hbm, agg_out, d_out,
                  asrc_loc, adst_loc, dloc, src_sb, dst_a, dst_b, p_sb,
                  rows, agg_sh,
                  g0, g1, g2, g3, s0, s1, s2, s3, ia, ib, iz):
    sem_g = (g0, g1, g2, g3)
    sem_s = (s0, s1, s2, s3)
    sem_i = (ia, ib)
    dst_p = (dst_a, dst_b)
    cid = lax.axis_index("c")
    sid = lax.axis_index("s")
    wid = sid
    base_e = wid * T_PER
    lo = cid * HALF

    pltpu.async_copy(asrc_hbm, asrc_loc, iz)
    pltpu.async_copy(adst_hbm, adst_loc, iz)
    pltpu.async_copy(shift_hbm, p_sb.at[pl.ds(0, 16)], iz)

    zero16 = jnp.zeros((16,), jnp.float32)
    lanes = lax.iota(jnp.int32, 16)

    def _idx_issue(sb, par):
        e0 = base_e + sb * SB_E
        pltpu.async_copy(src_hbm.at[pl.ds(e0, SB_E)],
                         src_sb.at[pl.ds(par * SB_E, SB_E)], sem_i[par])
        pltpu.async_copy(dst6_hbm.at[wid, sb], dst_p[par], sem_i[par])

    def _idx_wait(par):
        pltpu.make_async_copy(src_hbm.at[pl.ds(0, SB_E)],
                              src_sb.at[pl.ds(par * SB_E, SB_E)],
                              sem_i[par]).wait()
        pltpu.make_async_copy(dst6_hbm.at[0, 0], dst_p[par],
                              sem_i[par]).wait()

    _idx_issue(0, 0)
    _idx_issue(1, 1)

    def _zero_d(v, _):
        dloc[pl.ds(v * 16, 16)] = zero16
        return 0
    lax.fori_loop(0, A_PAD // 16, _zero_d, 0)

    def _zero_r(j, _):
        for k in range(8):
            rows[j, pl.ds(k * 16, 16)] = zero16
        return 0
    lax.fori_loop(0, NBUF * CHUNK, _zero_r, 0)
    for i in range(4):
        pltpu.async_copy(rows, agg_sh.at[pl.ds(sid * R_SL + i * 128, 128)],
                         iz)
    pltpu.async_copy(rows.at[pl.ds(0, R_SL - 512)],
                     agg_sh.at[pl.ds(sid * R_SL + 512, R_SL - 512)], iz)
    pltpu.make_async_copy(asrc_hbm, asrc_loc, iz).wait()
    pltpu.make_async_copy(adst_hbm, adst_loc, iz).wait()
    pltpu.make_async_copy(shift_hbm, p_sb.at[pl.ds(0, 16)], iz).wait()
    for i in range(4):
        pltpu.make_async_copy(
            rows, agg_sh.at[pl.ds(sid * R_SL, 128)], iz).wait()
    pltpu.make_async_copy(rows.at[pl.ds(0, R_SL - 512)],
                          agg_sh.at[pl.ds(sid * R_SL, R_SL - 512)],
                          iz).wait()
    shift = p_sb[pl.ds(0, 16)]
    plsc.subcore_barrier()

    def _gather(h, b, par):
        return pltpu.async_copy(
            xw_hbm.at[src_sb.at[pl.ds(par * SB_E + h * CHUNK, CHUNK)]],
            rows.at[pl.ds(b * CHUNK, CHUNK)], sem_g[b])

    def _scatter(h, b, par):
        return pltpu.async_copy(
            rows.at[pl.ds(b * CHUNK, CHUNK)],
            agg_sh.at[dst_p[par].at[h]], sem_s[b], add=True)

    def _pair_body(cp, _):
        for par in range(2):
            sb = 2 * cp + par
            e0 = base_e + sb * SB_E
            _idx_wait(par)
            gd = [None] * SBC
            sd = [None] * SBC
            gd[0] = _gather(0, 0, par)
            gd[1] = _gather(1, 1, par)

            dst_sb = dst_p[par]

            def _pa(j, _, _par=par, _dst=dst_sb, _e0=e0):
                for k in range(CHUNK // 16):
                    s16 = src_sb[pl.ds(_par * SB_E + j * CHUNK + k * 16, 16)]
                    d16 = _dst[j, pl.ds(k * 16, 16)]
                    vs = plsc.load_gather(asrc_loc, [s16])
                    vd = plsc.load_gather(adst_loc, [d16])
                    al = vs + vd
                    al = jnp.where(al > 0, al, NEG_SLOPE * al)
                    p = jnp.exp(al - shift)
                    eidx = _e0 + j * CHUNK + k * 16 + lanes
                    p = jnp.where(eidx < E_TOT, p, 0.0)
                    p_sb[pl.ds(j * CHUNK + k * 16, 16)] = p
                    de = d16 - lo
                    inhalf = jnp.logical_and(de >= 0, de < HALF)
                    plsc.addupdate_scatter(
                        dloc, [d16], jnp.where(inhalf, p, 0.0))
                    _dst[j, pl.ds(k * 16, 16)] = jnp.where(
                        inhalf, de, TRASH)
                return 0
            lax.fori_loop(0, SBC, _pa, 0)

            for h in range(SBC):
                b = h % NBUF
                gd[h].wait()
                if h >= 2:
                    sd[h - 2].wait()
                if h + 2 < SBC:
                    gd[h + 2] = _gather(h + 2, (h + 2) % NBUF, par)

                def _scale_grp(g, _, _h=h, _b=b):
                    p16 = p_sb[pl.ds(_h * CHUNK + g * 16, 16)]
                    for jj in range(16):
                        s = p16[jj]
                        row = _b * CHUNK + g * 16 + jj
                        for k in range(8):
                            rows[row, pl.ds(k * 16, 16)] = (
                                rows[row, pl.ds(k * 16, 16)] * s)
                    return 0
                lax.fori_loop(0, CHUNK // 16, _scale_grp, 0)
                sd[h] = _scatter(h, b, par)
            sd[SBC - 2].wait()
            sd[SBC - 1].wait()

            @pl.when(sb + 2 < SB_CNT)
            def _():
                _idx_issue(sb + 2, par)
        return 0
    lax.fori_loop(0, SB_CNT // 2, _pair_body, 0)
    plsc.subcore_barrier()

    pltpu.sync_copy(agg_sh.at[pl.ds(sid * A_SL, A_SL)],
                    agg_out.at[cid, pl.ds(sid * A_SL, A_SL)])
    pltpu.sync_copy(dloc, d_out.at[cid * NS + sid])


def kernel(x, edge_index, W, att_src, att_dst, bias):
    x = x.astype(jnp.float32)
    W = W.astype(jnp.float32)
    asv = att_src.reshape(1, DIM).astype(jnp.float32)
    adv = att_dst.reshape(1, DIM).astype(jnp.float32)

    xw, asr, adr = _tc_prep(x, W, asv, adv)
    a_src = asr[:, 0]
    a_dst = adr[:, 0]

    m = jnp.maximum(jnp.max(a_src) + jnp.max(a_dst), 0.0)
    shift16 = jnp.full((16,), m, jnp.float32)

    loops = jnp.arange(N_NODES, dtype=jnp.int32)
    pad = jnp.arange(N_PAD, dtype=jnp.int32) % N_NODES
    src = jnp.concatenate([edge_index[0].astype(jnp.int32), loops, pad])
    dst = jnp.concatenate([edge_index[1].astype(jnp.int32), loops, pad])
    dst6 = dst.reshape(NS, SB_CNT, SBC, CHUNK)

    agg, dpart = _get_sc_edge()(xw, a_src, a_dst, src, dst6, shift16)
    agg_full = jnp.concatenate([agg[0], agg[1][: A_PAD - HALF]], axis=0)

    b2 = bias.reshape(1, DIM).astype(jnp.float32)
    return _tc_post(agg_full, dpart, b2)[:N_NODES]


# speedup vs baseline: 1.0370x; 1.0370x over previous
import functools

import jax
import jax.numpy as jnp
from jax import lax
from jax.experimental import pallas as pl
from jax.experimental.pallas import tpu as pltpu
from jax.experimental.pallas import tpu_sc as plsc

N_NODES = 10000
DIM = 128
NEG_SLOPE = 0.2

NC = 1
NS = 16
NW = NC * NS
E_TOT = 320000 + N_NODES
CHUNK = 32
SBC = 6
SB_CNT = 108
SB_E = SBC * CHUNK
T_PER = SB_E * SB_CNT
E_PAD = T_PER * NW
N_PAD = E_PAD - E_TOT
D_PAD = 10240
A_PAD = 10112
R_SL = A_PAD // NS
NBUF = 4

ROW_BLK = 1000


def _tc_prep_body(x_ref, w_ref, asv_ref, adv_ref, xw_ref, asr_ref, adr_ref):
    xw = jnp.dot(x_ref[...], w_ref[...], preferred_element_type=jnp.float32)
    xw_ref[...] = xw
    asr_ref[...] = jnp.sum(xw * asv_ref[...], axis=1, keepdims=True)
    adr_ref[...] = jnp.sum(xw * adv_ref[...], axis=1, keepdims=True)


_tc_prep = pl.pallas_call(
    _tc_prep_body,
    grid=(N_NODES // ROW_BLK,),
    in_specs=[
        pl.BlockSpec((ROW_BLK, DIM), lambda i: (i, 0)),
        pl.BlockSpec((DIM, DIM), lambda i: (0, 0)),
        pl.BlockSpec((1, DIM), lambda i: (0, 0)),
        pl.BlockSpec((1, DIM), lambda i: (0, 0)),
    ],
    out_specs=[
        pl.BlockSpec((ROW_BLK, DIM), lambda i: (i, 0)),
        pl.BlockSpec((ROW_BLK, 1), lambda i: (i, 0)),
        pl.BlockSpec((ROW_BLK, 1), lambda i: (i, 0)),
    ],
    out_shape=[
        jax.ShapeDtypeStruct((N_NODES, DIM), jnp.float32),
        jax.ShapeDtypeStruct((N_NODES, 1), jnp.float32),
        jax.ShapeDtypeStruct((N_NODES, 1), jnp.float32),
    ],
)


def _tc_post_body(a0_ref, dp_ref, b_ref, out_ref):
    i = pl.program_id(0)
    dp = dp_ref[:, pl.ds(i * 128, 128)]
    d = jnp.sum(dp, axis=0)[:, None]
    h = a0_ref[...] / (d + 1e-16) + b_ref[...]
    out_ref[...] = h * jax.nn.sigmoid(h)


_tc_post = pl.pallas_call(
    _tc_post_body,
    grid=(A_PAD // 128,),
    in_specs=[
        pl.BlockSpec((128, DIM), lambda i: (i, 0)),
        pl.BlockSpec((NS, A_PAD), lambda i: (0, 0)),
        pl.BlockSpec((1, DIM), lambda i: (0, 0)),
    ],
    out_specs=pl.BlockSpec((128, DIM), lambda i: (i, 0)),
    out_shape=jax.ShapeDtypeStruct((A_PAD, DIM), jnp.float32),
)


@functools.cache
def _get_sc_edge():
  return pl.kernel(
    _sc_edge_body,
    out_type=[
        jax.ShapeDtypeStruct((A_PAD, DIM), jnp.float32),
        jax.ShapeDtypeStruct((NS, A_PAD), jnp.float32),
    ],
    mesh=plsc.VectorSubcoreMesh(core_axis_name="c", subcore_axis_name="s",
                                num_cores=NC, num_subcores=NS),
    compiler_params=pltpu.CompilerParams(needs_layout_passes=False),
    scratch_types=[
        pltpu.VMEM((N_NODES,), jnp.float32),
        pltpu.VMEM((N_NODES,), jnp.float32),
        pltpu.VMEM((A_PAD,), jnp.float32),
        pltpu.VMEM((2 * SB_E,), jnp.int32),
        pltpu.VMEM((SBC, CHUNK), jnp.int32),
        pltpu.VMEM((SBC, CHUNK), jnp.int32),
        pltpu.VMEM((SB_E,), jnp.float32),
        pltpu.VMEM((NBUF * CHUNK, DIM), jnp.float32),
        pltpu.VMEM_SHARED((A_PAD, DIM), jnp.float32),
        pltpu.SemaphoreType.DMA,
        pltpu.SemaphoreType.DMA,
        pltpu.SemaphoreType.DMA,
        pltpu.SemaphoreType.DMA,
        pltpu.SemaphoreType.DMA,
        pltpu.SemaphoreType.DMA,
        pltpu.SemaphoreType.DMA,
        pltpu.SemaphoreType.DMA,
        pltpu.SemaphoreType.DMA,
        pltpu.SemaphoreType.DMA,
        pltpu.SemaphoreType.DMA,
    ],
  )


def _sc_edge_body(xw_hbm, asrc_hbm, adst_hbm, src_hbm, dst6_hbm,
                  shift_hbm, agg_out, d_out,
                  asrc_loc, adst_loc, dloc, src_sb, dst_a, dst_b, p_sb,
                  rows, agg_sh,
                  g0, g1, g2, g3, s0, s1, s2, s3, ia, ib, iz):
    sem_g = (g0, g1, g2, g3)
    sem_s = (s0, s1, s2, s3)
    sem_i = (ia, ib)
    dst_p = (dst_a, dst_b)
    sid = lax.axis_index("s")
    wid = sid
    base_e = wid * T_PER

    pltpu.async_copy(asrc_hbm, asrc_loc, iz)
    pltpu.async_copy(adst_hbm, adst_loc, iz)
    pltpu.async_copy(shift_hbm, p_sb.at[pl.ds(0, 16)], iz)

    zero16 = jnp.zeros((16,), jnp.float32)
    lanes = lax.iota(jnp.int32, 16)

    def _idx_issue(sb, par):
        e0 = base_e + sb * SB_E
        pltpu.async_copy(src_hbm.at[pl.ds(e0, SB_E)],
                         src_sb.at[pl.ds(par * SB_E, SB_E)], sem_i[par])
        pltpu.async_copy(dst6_hbm.at[wid, sb], dst_p[par], sem_i[par])

    def _idx_wait(par):
        pltpu.make_async_copy(src_hbm.at[pl.ds(0, SB_E)],
                              src_sb.at[pl.ds(par * SB_E, SB_E)],
                              sem_i[par]).wait()
        pltpu.make_async_copy(dst6_hbm.at[0, 0], dst_p[par],
                              sem_i[par]).wait()

    _idx_issue(0, 0)
    _idx_issue(1, 1)

    def _zero_d(v, _):
        dloc[pl.ds(v * 16, 16)] = zero16
        return 0
    lax.fori_loop(0, A_PAD // 16, _zero_d, 0)

    def _zero_r(j, _):
        for k in range(8):
            rows[j, pl.ds(k * 16, 16)] = zero16
        return 0
    lax.fori_loop(0, NBUF * CHUNK, _zero_r, 0)
    for i in range(4):
        pltpu.async_copy(rows, agg_sh.at[pl.ds(sid * R_SL + i * 128, 128)],
                         iz)
    pltpu.async_copy(rows.at[pl.ds(0, R_SL - 512)],
                     agg_sh.at[pl.ds(sid * R_SL + 512, R_SL - 512)], iz)
    pltpu.make_async_copy(asrc_hbm, asrc_loc, iz).wait()
    pltpu.make_async_copy(adst_hbm, adst_loc, iz).wait()
    pltpu.make_async_copy(shift_hbm, p_sb.at[pl.ds(0, 16)], iz).wait()
    for i in range(4):
        pltpu.make_async_copy(
            rows, agg_sh.at[pl.ds(sid * R_SL, 128)], iz).wait()
    pltpu.make_async_copy(rows.at[pl.ds(0, R_SL - 512)],
                          agg_sh.at[pl.ds(sid * R_SL, R_SL - 512)],
                          iz).wait()
    shift = p_sb[pl.ds(0, 16)]
    plsc.subcore_barrier()

    def _gather(h, b, par):
        return pltpu.async_copy(
            xw_hbm.at[src_sb.at[pl.ds(par * SB_E + h * CHUNK, CHUNK)]],
            rows.at[pl.ds(b * CHUNK, CHUNK)], sem_g[b])

    def _scatter(h, b, par):
        return pltpu.async_copy(
            rows.at[pl.ds(b * CHUNK, CHUNK)],
            agg_sh.at[dst_p[par].at[h]], sem_s[b], add=True)

    def _pair_body(cp, _):
        for par in range(2):
            sb = 2 * cp + par
            e0 = base_e + sb * SB_E
            _idx_wait(par)
            gd = [None] * SBC
            sd = [None] * SBC
            gd[0] = _gather(0, 0, par)
            gd[1] = _gather(1, 1, par)

            dst_sb = dst_p[par]

            def _pa(j, _, _par=par, _dst=dst_sb, _e0=e0):
                for k in range(CHUNK // 16):
                    s16 = src_sb[pl.ds(_par * SB_E + j * CHUNK + k * 16, 16)]
                    d16 = _dst[j, pl.ds(k * 16, 16)]
                    vs = plsc.load_gather(asrc_loc, [s16])
                    vd = plsc.load_gather(adst_loc, [d16])
                    al = vs + vd
                    al = jnp.where(al > 0, al, NEG_SLOPE * al)
                    p = jnp.exp(al - shift)
                    eidx = _e0 + j * CHUNK + k * 16 + lanes
                    p = jnp.where(eidx < E_TOT, p, 0.0)
                    p_sb[pl.ds(j * CHUNK + k * 16, 16)] = p
                    plsc.addupdate_scatter(dloc, [d16], p)
                return 0
            lax.fori_loop(0, SBC, _pa, 0)

            for h in range(SBC):
                b = h % NBUF
                gd[h].wait()
                if h >= 2:
                    sd[h - 2].wait()
                if h + 2 < SBC:
                    gd[h + 2] = _gather(h + 2, (h + 2) % NBUF, par)

                def _scale_grp(g, _, _h=h, _b=b):
                    p16 = p_sb[pl.ds(_h * CHUNK + g * 16, 16)]
                    for jj in range(16):
                        s = p16[jj]
                        row = _b * CHUNK + g * 16 + jj
                        for k in range(8):
                            rows[row, pl.ds(k * 16, 16)] = (
                                rows[row, pl.ds(k * 16, 16)] * s)
                    return 0
                lax.fori_loop(0, CHUNK // 16, _scale_grp, 0)
                sd[h] = _scatter(h, b, par)
            sd[SBC - 2].wait()
            sd[SBC - 1].wait()

            @pl.when(sb + 2 < SB_CNT)
            def _():
                _idx_issue(sb + 2, par)
        return 0
    lax.fori_loop(0, SB_CNT // 2, _pair_body, 0)
    plsc.subcore_barrier()

    pltpu.sync_copy(agg_sh.at[pl.ds(sid * R_SL, R_SL)],
                    agg_out.at[pl.ds(sid * R_SL, R_SL)])
    pltpu.sync_copy(dloc, d_out.at[sid])


def kernel(x, edge_index, W, att_src, att_dst, bias):
    x = x.astype(jnp.float32)
    W = W.astype(jnp.float32)
    asv = att_src.reshape(1, DIM).astype(jnp.float32)
    adv = att_dst.reshape(1, DIM).astype(jnp.float32)

    xw, asr, adr = _tc_prep(x, W, asv, adv)
    a_src = asr[:, 0]
    a_dst = adr[:, 0]

    m = jnp.maximum(jnp.max(a_src) + jnp.max(a_dst), 0.0)
    shift16 = jnp.full((16,), m, jnp.float32)

    loops = jnp.arange(N_NODES, dtype=jnp.int32)
    pad = jnp.arange(N_PAD, dtype=jnp.int32) % N_NODES
    src = jnp.concatenate([edge_index[0].astype(jnp.int32), loops, pad])
    dst = jnp.concatenate([edge_index[1].astype(jnp.int32), loops, pad])
    dst6 = dst.reshape(NW, SB_CNT, SBC, CHUNK)

    agg, dpart = _get_sc_edge()(xw, a_src, a_dst, src, dst6, shift16)

    b2 = bias.reshape(1, DIM).astype(jnp.float32)
    return _tc_post(agg, dpart, b2)[:N_NODES]
